# baseline (speedup 1.0000x reference)
import jax
import jax.numpy as jnp
from jax.experimental import pallas as pl

_BATCH = 4
_ROWS = 8192
_DIM = 256
_BLOCK = 4096


def _bcast_body(w_ref, out_ref):
    out_ref[...] = jnp.broadcast_to(w_ref[...][None], (_BATCH, _BLOCK, _DIM))


def kernel(tokens, W):
    del tokens
    grid = (_ROWS // _BLOCK,)
    return pl.pallas_call(
        _bcast_body,
        grid=grid,
        in_specs=[pl.BlockSpec((_BLOCK, _DIM), lambda i: (i, 0))],
        out_specs=pl.BlockSpec((_BATCH, _BLOCK, _DIM), lambda i: (0, i, 0)),
        out_shape=jax.ShapeDtypeStruct((_BATCH, _ROWS, _DIM), jnp.float32),
    )(W)
